# baseline (speedup 1.0000x reference)
import functools

import jax
import jax.numpy as jnp
from jax import lax
from jax.experimental import pallas as pl
from jax.experimental.pallas import tpu as pltpu
from jax.experimental.pallas import tpu_sc as plsc

_B = 16384
_A = 128
_NC = 2
_NS = 16
_NW = _NC * _NS
_BPW = _B // _NW
_CHUNK = 128
_NCHUNK = _BPW // _CHUNK
_L = 16
_NCHAIN = 8

_mesh = plsc.VectorSubcoreMesh(core_axis_name="c", subcore_axis_name="s")


@functools.partial(
    pl.kernel,
    out_type=jax.ShapeDtypeStruct((_B,), jnp.int32),
    mesh=_mesh,
    compiler_params=pltpu.CompilerParams(needs_layout_passes=False),
    scratch_types=[
        pltpu.VMEM((_BPW,), jnp.int32),
        pltpu.VMEM((_CHUNK, _A), jnp.float32),
        pltpu.VMEM((_CHUNK, _A), jnp.float32),
        pltpu.VMEM((_BPW,), jnp.int32),
        pltpu.SemaphoreType.DMA,
        pltpu.SemaphoreType.DMA,
    ],
)
def _argmax_gather(states_hbm, policy_hbm, out_hbm,
                   idx_v, buf0, buf1, out_v, sem0, sem1):
    wid = lax.axis_index("s") * _NC + lax.axis_index("c")
    base = wid * _BPW
    pltpu.sync_copy(states_hbm.at[pl.ds(base, _BPW)], idx_v)

    bufs = (buf0, buf1)
    sems = (sem0, sem1)

    def start(k):
        return pltpu.async_copy(
            policy_hbm.at[idx_v.at[pl.ds(k * _CHUNK, _CHUNK)]],
            bufs[k % 2], sems[k % 2])

    def compute(k):
        buf = bufs[k % 2]

        def group_body(g, _):
            row_ids = lax.iota(jnp.int32, _L) + g * _L
            lane = lax.iota(jnp.int32, _L)
            m = None
            wb = None
            for r in range(0, _NCHAIN, 4):
                ph = lane
                maxes = [
                    plsc.load_gather(buf, [row_ids, ph + j * _L])
                    for j in range(r, r + 4)
                ]
                for _t in range(1, _L):
                    ph = (ph + 1) & (_L - 1)
                    for jj, j in enumerate(range(r, r + 4)):
                        maxes[jj] = jnp.maximum(
                            maxes[jj],
                            plsc.load_gather(buf, [row_ids, ph + j * _L]))
                for jj, j in enumerate(range(r, r + 4)):
                    if m is None:
                        m, wb = maxes[jj], jnp.zeros((_L,), jnp.int32)
                    else:
                        gt = maxes[jj] > m
                        m = jnp.where(gt, maxes[jj], m)
                        wb = jnp.where(gt, j * _L, wb)

            ph = lane
            v = plsc.load_gather(buf, [row_ids, wb + ph])
            mc = jnp.where(v == m, wb + ph, _A * 2)
            for _t in range(1, _L):
                ph = (ph + 1) & (_L - 1)
                col = wb + ph
                v = plsc.load_gather(buf, [row_ids, col])
                mc = jnp.minimum(mc, jnp.where(v == m, col, _A * 2))

            out_v[pl.ds(k * _CHUNK + g * _L, _L)] = mc
            return 0

        lax.fori_loop(0, _CHUNK // _L, group_body, 0)

    cp = start(0)
    for k in range(_NCHUNK):
        nxt = start(k + 1) if k + 1 < _NCHUNK else None
        cp.wait()
        compute(k)
        cp = nxt

    pltpu.sync_copy(out_v, out_hbm.at[pl.ds(base, _BPW)])


def kernel(states, policy):
    return _argmax_gather(states.astype(jnp.int32), policy)


# speedup vs baseline: 1.0714x; 1.0714x over previous
import functools

import jax
import jax.numpy as jnp
from jax import lax
from jax.experimental import pallas as pl
from jax.experimental.pallas import tpu as pltpu
from jax.experimental.pallas import tpu_sc as plsc

_B = 16384
_A = 128
_NC = 2
_NS = 16
_NW = _NC * _NS
_BPW = _B // _NW
_CHUNK = 128
_NCHUNK = _BPW // _CHUNK
_L = 16
_NCHAIN = 8

_mesh = plsc.VectorSubcoreMesh(core_axis_name="c", subcore_axis_name="s")


@functools.partial(
    pl.kernel,
    out_type=jax.ShapeDtypeStruct((_B,), jnp.int32),
    mesh=_mesh,
    compiler_params=pltpu.CompilerParams(needs_layout_passes=False),
    scratch_types=[
        pltpu.VMEM((_BPW,), jnp.int32),
        pltpu.VMEM((_CHUNK, _A), jnp.float32),
        pltpu.VMEM((_CHUNK, _A), jnp.float32),
        pltpu.VMEM((_BPW,), jnp.int32),
        pltpu.SemaphoreType.DMA,
        pltpu.SemaphoreType.DMA,
    ],
)
def _argmax_gather(states_hbm, policy_hbm, out_hbm,
                   idx_v, buf0, buf1, out_v, sem0, sem1):
    wid = lax.axis_index("s") * _NC + lax.axis_index("c")
    base = wid * _BPW
    pltpu.sync_copy(states_hbm.at[pl.ds(base, _BPW)], idx_v)

    bufs = (buf0, buf1)
    sems = (sem0, sem1)

    def start(k):
        return pltpu.async_copy(
            policy_hbm.at[idx_v.at[pl.ds(k * _CHUNK, _CHUNK)]],
            bufs[k % 2], sems[k % 2])

    def compute(k):
        buf = bufs[k % 2]

        def group_body(g, _):
            row_ids = lax.iota(jnp.int32, _L) + g * _L
            def sweep(cols, bvs, bis):
                ncols, nbvs, nbis = [], [], []
                for j in range(_NCHAIN):
                    col = (cols[j] + 1) & (_A - 1)
                    v = plsc.load_gather(buf, [row_ids, col])
                    upd = (v > bvs[j]) | ((v == bvs[j]) & (col < bis[j]))
                    ncols.append(col)
                    nbvs.append(jnp.where(upd, v, bvs[j]))
                    nbis.append(jnp.where(upd, col, bis[j]))
                return ncols, nbvs, nbis

            cols = [lax.iota(jnp.int32, _L) + j * (_A // _NCHAIN)
                    for j in range(_NCHAIN)]
            bvs = [plsc.load_gather(buf, [row_ids, c]) for c in cols]
            bis = list(cols)

            def step_body(_s, carry):
                cols, bvs, bis = carry
                for _u in range(3):
                    cols, bvs, bis = sweep(cols, bvs, bis)
                return cols, bvs, bis

            cols, bvs, bis = lax.fori_loop(
                0, (_A // _NCHAIN - 1) // 3, step_body, (cols, bvs, bis))

            step = 1
            while step < _NCHAIN:
                for j in range(0, _NCHAIN, 2 * step):
                    v, c = bvs[j + step], bis[j + step]
                    upd = (v > bvs[j]) | ((v == bvs[j]) & (c < bis[j]))
                    bvs[j] = jnp.where(upd, v, bvs[j])
                    bis[j] = jnp.where(upd, c, bis[j])
                step *= 2
            out_v[pl.ds(k * _CHUNK + g * _L, _L)] = bis[0]
            return 0

        lax.fori_loop(0, _CHUNK // _L, group_body, 0)

    cp = start(0)
    for k in range(_NCHUNK):
        nxt = start(k + 1) if k + 1 < _NCHUNK else None
        cp.wait()
        compute(k)
        cp = nxt

    pltpu.sync_copy(out_v, out_hbm.at[pl.ds(base, _BPW)])


def kernel(states, policy):
    return _argmax_gather(states.astype(jnp.int32), policy)
